# baseline (speedup 1.0000x reference)
import jax
import jax.numpy as jnp
from jax import lax
from jax.experimental import pallas as pl
from jax.experimental.pallas import tpu as pltpu
from jax.experimental.pallas import tpu_sc as plsc

N = 16384
M_TOTAL = 256
S = 512
C = 128
D = C + 3
DP = 128
L = 16
NC = 2
NS = 16
NW = NC * NS
BOXES_PER_W = M_TOTAL // NW
SCAN_STEPS = N // L
CAP = S
CH = 64
NCH = S // CH


def _sc_body(coords_hbm, tailc_hbm, boxtab_hbm, tablea_hbm, out_hbm,
             coords_v, tailc_v, boxtab_v, compact_v, pidx_v,
             rows0_v, rows1_v, tail_v, gsem0, gsem1, wsem):
    wid = lax.axis_index("s") * NC + lax.axis_index("c")
    b = wid // NS
    pltpu.sync_copy(coords_hbm.at[pl.ds(b * 3 * N, 3 * N)], coords_v)
    pltpu.sync_copy(tailc_hbm.at[pl.ds(b * 3 * N, 3 * N)], tailc_v)
    pltpu.sync_copy(boxtab_hbm.at[pl.ds(wid * BOXES_PER_W * 16,
                                        BOXES_PER_W * 16)], boxtab_v)
    iota = lax.iota(jnp.int32, L)
    zi = jnp.zeros((L,), jnp.int32)
    mall = iota >= 0
    rows = (rows0_v, rows1_v)
    gsem = (gsem0, gsem1)

    def box_body(k, _carry):
        m = wid * BOXES_PER_W + k
        row_v = boxtab_v[pl.ds(k * 16, 16)]
        cx = row_v[0]
        cy = row_v[1]
        czb = row_v[2]
        czc = row_v[3]
        hx = row_v[4]
        hy = row_v[5]
        hz = row_v[6]
        ca = row_v[7]
        sa = row_v[8]

        compact_v[pl.ds(0, L)] = zi

        def scan_body(i, cnt):
            base = i * L
            xv = coords_v[pl.ds(base, L)]
            yv = coords_v[pl.ds(N + base, L)]
            zv = coords_v[pl.ds(2 * N + base, L)]
            sx = xv - cx
            sy = yv - cy
            sz = zv - czc
            lx = ca * sx + sa * sy
            ly = ca * sy - sa * sx
            msk = ((jnp.abs(sz) <= hz) & (jnp.abs(lx) <= hx)
                   & (jnp.abs(ly) <= hy))
            pcnt = plsc.all_reduce_population_count(msk)
            tot = pcnt[0]

            @pl.when(tot > 0)
            def _compact():
                mi = msk.astype(jnp.int32)
                pc = jnp.cumsum(mi)
                off = jnp.minimum(cnt, CAP)
                pos = pc - 1 + off
                plsc.store_scatter(compact_v, [pos], iota + base, mask=msk)

            return cnt + tot

        cnt = lax.fori_loop(0, SCAN_STEPS, scan_body, jnp.int32(0),
                            unroll=4)

        denom = jnp.maximum(cnt, 1)
        ne_i = (cnt > 0).astype(jnp.int32)
        ne_f = (cnt > 0).astype(jnp.float32)
        boff = b * N

        def pick_body(g, _c):
            jv = iota + g * L
            sel = jv % denom
            lidx = plsc.load_gather(compact_v, [sel]) * ne_i
            plsc.store_scatter(pidx_v, [jv], lidx + boff, mask=mall)
            return _c

        lax.fori_loop(0, S // L, pick_body, jnp.int32(0))

        def fire_gather(h):
            s = h % 2
            return pltpu.async_copy(
                tablea_hbm.at[pidx_v.at[pl.ds(h * CH, CH)]],
                rows[s], gsem[s])

        def do_patch(hp):
            sp = hp % 2

            def patch_body(g, _c):
                jl = iota + g * L
                jv = jl + hp * CH
                sel = jv % denom
                lidx = plsc.load_gather(compact_v, [sel]) * ne_i
                px = plsc.load_gather(coords_v, [lidx])
                py = plsc.load_gather(coords_v, [lidx + N])
                pz = plsc.load_gather(coords_v, [lidx + 2 * N])
                rx = px - cx
                ry = py - cy
                ox = (rx * ca + ry * sa) * ne_f
                oy = (ry * ca - rx * sa) * ne_f
                oz = (pz - czb) * ne_f
                plsc.store_scatter(rows[sp], [jl, zi], ox, mask=mall)
                plsc.store_scatter(rows[sp], [jl, zi + 1], oy, mask=mall)
                plsc.store_scatter(rows[sp], [jl, zi + 2], oz, mask=mall)
                for c in range(3):
                    tv = plsc.load_gather(tailc_v, [lidx + c * N]) * ne_f
                    plsc.store_scatter(tail_v, [jl, zi + c], tv,
                                       mask=mall)
                return _c

            lax.fori_loop(0, CH // L, patch_body, jnp.int32(0))

            @pl.when(cnt == 0)
            def _zero_rows():
                zfull = jnp.zeros((L,), jnp.float32)

                def zr(i, _c):
                    for c8 in range(C // L):
                        rows[sp][i, pl.ds(c8 * L, L)] = zfull
                    return _c
                lax.fori_loop(0, CH, zr, jnp.int32(0))

            wm = pltpu.async_copy(
                rows[sp],
                out_hbm.at[m, pl.ds(hp * CH, CH), pl.ds(0, C)], wsem)
            wt = pltpu.async_copy(
                tail_v,
                out_hbm.at[m, pl.ds(hp * CH, CH), pl.ds(C, 3)], wsem)
            return [wm, wt]

        pending_w = {0: [], 1: []}
        g_prev = fire_gather(0)
        for h in range(1, NCH + 1):
            s = h % 2
            for cp in pending_w[s]:
                cp.wait()
            pending_w[s] = []
            g_new = fire_gather(h) if h < NCH else None
            g_prev.wait()
            pending_w[(h - 1) % 2] = do_patch(h - 1)
            g_prev = g_new
        for s in range(2):
            for cp in pending_w[s]:
                cp.wait()
        return _carry

    lax.fori_loop(0, BOXES_PER_W, box_body, jnp.int32(0))


@jax.jit
def _run_sc(coords_flat, tailc_flat, boxtab_flat, tablea):
    mesh = plsc.VectorSubcoreMesh(core_axis_name="c", subcore_axis_name="s")
    f = pl.kernel(
        _sc_body,
        out_type=jax.ShapeDtypeStruct((M_TOTAL, S, D), jnp.float32),
        mesh=mesh,
        scratch_types=[
            pltpu.VMEM((3 * N,), jnp.float32),
            pltpu.VMEM((3 * N,), jnp.float32),
            pltpu.VMEM((BOXES_PER_W * 16,), jnp.float32),
            pltpu.VMEM((CAP + L,), jnp.int32),
            pltpu.VMEM((S,), jnp.int32),
            pltpu.VMEM((CH, DP), jnp.float32),
            pltpu.VMEM((CH, DP), jnp.float32),
            pltpu.VMEM((CH, 3), jnp.float32),
            pltpu.SemaphoreType.DMA,
            pltpu.SemaphoreType.DMA,
            pltpu.SemaphoreType.DMA,
        ],
        compiler_params=pltpu.CompilerParams(needs_layout_passes=False),
    )
    return f(coords_flat, tailc_flat, boxtab_flat, tablea)


def kernel(feats, coordinate, batch_inds, rois):
    del batch_inds
    B = coordinate.shape[0]
    r = rois[:, 1:]
    cx, cy, cz = r[:, 0], r[:, 1], r[:, 2]
    dx, dy, dz = r[:, 3], r[:, 4], r[:, 5]
    rz = r[:, 6]
    zcol = jnp.zeros_like(cx)
    boxtab = jnp.stack(
        [cx, cy, cz, cz + dz / 2.0, dx / 2.0, dy / 2.0, dz / 2.0,
         jnp.cos(rz), jnp.sin(rz), zcol, zcol, zcol, zcol, zcol, zcol, zcol],
        axis=-1)
    coords_flat = coordinate.transpose(0, 2, 1).reshape(B * 3 * N)
    tailc_flat = feats[:, :, C - 3:].transpose(0, 2, 1).reshape(B * 3 * N)
    tablea = jnp.concatenate([coordinate, feats[:, :, :C - 3]],
                             axis=-1).reshape(B * N, DP)
    return _run_sc(coords_flat, tailc_flat, boxtab.reshape(-1), tablea)


# speedup vs baseline: 1.0447x; 1.0447x over previous
import jax
import jax.numpy as jnp
from jax import lax
from jax.experimental import pallas as pl
from jax.experimental.pallas import tpu as pltpu
from jax.experimental.pallas import tpu_sc as plsc

N = 16384
M_TOTAL = 256
S = 512
C = 128
D = C + 3
DP = 128
L = 16
NC = 2
NS = 16
NW = NC * NS
BOXES_PER_W = M_TOTAL // NW
SCAN_STEPS = N // L
CAP = S
CH = 64
NCH = S // CH


def _sc_body(coords_hbm, tailc_hbm, boxtab_hbm, tablea_hbm, out_hbm,
             coords_v, tailc_v, boxtab_v, compact_v, pidx_v,
             rows0_v, rows1_v, tail_v, gsem0, gsem1, wsem):
    wid = lax.axis_index("s") * NC + lax.axis_index("c")
    b = wid // NS
    pltpu.sync_copy(coords_hbm.at[pl.ds(b * 3 * N, 3 * N)], coords_v)
    pltpu.sync_copy(tailc_hbm.at[pl.ds(b * 3 * N, 3 * N)], tailc_v)
    pltpu.sync_copy(boxtab_hbm.at[pl.ds(wid * BOXES_PER_W * 16,
                                        BOXES_PER_W * 16)], boxtab_v)
    iota = lax.iota(jnp.int32, L)
    zi = jnp.zeros((L,), jnp.int32)
    mall = iota >= 0
    rows = (rows0_v, rows1_v)
    gsem = (gsem0, gsem1)

    def box_body(k, _carry):
        m = wid * BOXES_PER_W + k
        row_v = boxtab_v[pl.ds(k * 16, 16)]
        cx = row_v[0]
        cy = row_v[1]
        czb = row_v[2]
        czc = row_v[3]
        hx = row_v[4]
        hy = row_v[5]
        hz = row_v[6]
        ca = row_v[7]
        sa = row_v[8]

        compact_v[pl.ds(0, L)] = zi

        def scan_body(i, cnt):
            base = i * L
            xv = coords_v[pl.ds(base, L)]
            yv = coords_v[pl.ds(N + base, L)]
            zv = coords_v[pl.ds(2 * N + base, L)]
            sx = xv - cx
            sy = yv - cy
            sz = zv - czc
            lx = ca * sx + sa * sy
            ly = ca * sy - sa * sx
            msk = ((jnp.abs(sz) <= hz) & (jnp.abs(lx) <= hx)
                   & (jnp.abs(ly) <= hy))
            mi = msk.astype(jnp.int32)
            pc = jnp.cumsum(mi)
            tot = plsc.all_reduce_population_count(msk)[0]
            off = jnp.minimum(cnt, CAP)
            pos = pc - 1 + off
            plsc.store_scatter(compact_v, [pos], iota + base, mask=msk)
            return cnt + tot

        cnt = lax.fori_loop(0, SCAN_STEPS, scan_body, jnp.int32(0),
                            unroll=4)

        denom = jnp.maximum(cnt, 1)
        ne_i = (cnt > 0).astype(jnp.int32)
        ne_f = (cnt > 0).astype(jnp.float32)
        boff = b * N

        def pick_body(g, _c):
            jv = iota + g * L
            sel = jv % denom
            lidx = plsc.load_gather(compact_v, [sel]) * ne_i
            plsc.store_scatter(pidx_v, [jv], lidx + boff, mask=mall)
            return _c

        lax.fori_loop(0, S // L, pick_body, jnp.int32(0))

        def fire_gather(h):
            s = h % 2
            return pltpu.async_copy(
                tablea_hbm.at[pidx_v.at[pl.ds(h * CH, CH)]],
                rows[s], gsem[s])

        def do_patch(hp):
            sp = hp % 2

            def patch_body(g, _c):
                jl = iota + g * L
                jv = jl + hp * CH
                sel = jv % denom
                lidx = plsc.load_gather(compact_v, [sel]) * ne_i
                px = plsc.load_gather(coords_v, [lidx])
                py = plsc.load_gather(coords_v, [lidx + N])
                pz = plsc.load_gather(coords_v, [lidx + 2 * N])
                rx = px - cx
                ry = py - cy
                ox = (rx * ca + ry * sa) * ne_f
                oy = (ry * ca - rx * sa) * ne_f
                oz = (pz - czb) * ne_f
                plsc.store_scatter(rows[sp], [jl, zi], ox, mask=mall)
                plsc.store_scatter(rows[sp], [jl, zi + 1], oy, mask=mall)
                plsc.store_scatter(rows[sp], [jl, zi + 2], oz, mask=mall)
                for c in range(3):
                    tv = plsc.load_gather(tailc_v, [lidx + c * N]) * ne_f
                    plsc.store_scatter(tail_v, [jl, zi + c], tv,
                                       mask=mall)
                return _c

            lax.fori_loop(0, CH // L, patch_body, jnp.int32(0))

            @pl.when(cnt == 0)
            def _zero_rows():
                zfull = jnp.zeros((L,), jnp.float32)

                def zr(i, _c):
                    for c8 in range(C // L):
                        rows[sp][i, pl.ds(c8 * L, L)] = zfull
                    return _c
                lax.fori_loop(0, CH, zr, jnp.int32(0))

            wm = pltpu.async_copy(
                rows[sp],
                out_hbm.at[m, pl.ds(hp * CH, CH), pl.ds(0, C)], wsem)
            wt = pltpu.async_copy(
                tail_v,
                out_hbm.at[m, pl.ds(hp * CH, CH), pl.ds(C, 3)], wsem)
            return [wm, wt]

        pending_w = {0: [], 1: []}
        g_prev = fire_gather(0)
        for h in range(1, NCH + 1):
            s = h % 2
            for cp in pending_w[s]:
                cp.wait()
            pending_w[s] = []
            g_new = fire_gather(h) if h < NCH else None
            g_prev.wait()
            pending_w[(h - 1) % 2] = do_patch(h - 1)
            g_prev = g_new
        for s in range(2):
            for cp in pending_w[s]:
                cp.wait()
        return _carry

    lax.fori_loop(0, BOXES_PER_W, box_body, jnp.int32(0))


@jax.jit
def _run_sc(coords_flat, tailc_flat, boxtab_flat, tablea):
    mesh = plsc.VectorSubcoreMesh(core_axis_name="c", subcore_axis_name="s")
    f = pl.kernel(
        _sc_body,
        out_type=jax.ShapeDtypeStruct((M_TOTAL, S, D), jnp.float32),
        mesh=mesh,
        scratch_types=[
            pltpu.VMEM((3 * N,), jnp.float32),
            pltpu.VMEM((3 * N,), jnp.float32),
            pltpu.VMEM((BOXES_PER_W * 16,), jnp.float32),
            pltpu.VMEM((CAP + L,), jnp.int32),
            pltpu.VMEM((S,), jnp.int32),
            pltpu.VMEM((CH, DP), jnp.float32),
            pltpu.VMEM((CH, DP), jnp.float32),
            pltpu.VMEM((CH, 3), jnp.float32),
            pltpu.SemaphoreType.DMA,
            pltpu.SemaphoreType.DMA,
            pltpu.SemaphoreType.DMA,
        ],
        compiler_params=pltpu.CompilerParams(needs_layout_passes=False),
    )
    return f(coords_flat, tailc_flat, boxtab_flat, tablea)


def kernel(feats, coordinate, batch_inds, rois):
    del batch_inds
    B = coordinate.shape[0]
    r = rois[:, 1:]
    cx, cy, cz = r[:, 0], r[:, 1], r[:, 2]
    dx, dy, dz = r[:, 3], r[:, 4], r[:, 5]
    rz = r[:, 6]
    zcol = jnp.zeros_like(cx)
    boxtab = jnp.stack(
        [cx, cy, cz, cz + dz / 2.0, dx / 2.0, dy / 2.0, dz / 2.0,
         jnp.cos(rz), jnp.sin(rz), zcol, zcol, zcol, zcol, zcol, zcol, zcol],
        axis=-1)
    coords_flat = coordinate.transpose(0, 2, 1).reshape(B * 3 * N)
    tailc_flat = feats[:, :, C - 3:].transpose(0, 2, 1).reshape(B * 3 * N)
    tablea = jnp.concatenate([coordinate, feats[:, :, :C - 3]],
                             axis=-1).reshape(B * N, DP)
    return _run_sc(coords_flat, tailc_flat, boxtab.reshape(-1), tablea)


# speedup vs baseline: 1.3051x; 1.2492x over previous
import jax
import jax.numpy as jnp
from jax import lax
from jax.experimental import pallas as pl
from jax.experimental.pallas import tpu as pltpu
from jax.experimental.pallas import tpu_sc as plsc

N = 16384
M_TOTAL = 256
S = 512
C = 128
D = C + 3
DP = 128
L = 16
NC = 2
NS = 16
NW = NC * NS
BOXES_PER_W = M_TOTAL // NW
SCAN_STEPS = N // L
CAP = S
CH = 64
NCH = S // CH


def _sc_body(coords_hbm, tailc_hbm, boxtab_hbm, tablea_hbm, out_hbm,
             coords_v, tailc_v, boxtab_v, compact_v, pidx_v,
             rows0_v, rows1_v, tail_v, gsem0, gsem1, wsem):
    wid = lax.axis_index("s") * NC + lax.axis_index("c")
    b = wid // NS
    pltpu.sync_copy(coords_hbm.at[pl.ds(b * 3 * N, 3 * N)], coords_v)
    pltpu.sync_copy(tailc_hbm.at[pl.ds(b * 3 * N, 3 * N)], tailc_v)
    pltpu.sync_copy(boxtab_hbm.at[pl.ds(wid * BOXES_PER_W * 16,
                                        BOXES_PER_W * 16)], boxtab_v)
    iota = lax.iota(jnp.int32, L)
    zi = jnp.zeros((L,), jnp.int32)
    mall = iota >= 0
    rows = (rows0_v, rows1_v)
    gsem = (gsem0, gsem1)

    def box_body(k, _carry):
        m = wid * BOXES_PER_W + k
        row_v = boxtab_v[pl.ds(k * 16, 16)]
        cx = row_v[0]
        cy = row_v[1]
        czb = row_v[2]
        czc = row_v[3]
        hx = row_v[4]
        hy = row_v[5]
        hz = row_v[6]
        ca = row_v[7]
        sa = row_v[8]

        compact_v[pl.ds(0, L)] = zi

        def scan_body(i, cnt):
            base = i * L
            xv = coords_v[pl.ds(base, L)]
            yv = coords_v[pl.ds(N + base, L)]
            zv = coords_v[pl.ds(2 * N + base, L)]
            sx = xv - cx
            sy = yv - cy
            sz = zv - czc
            lx = ca * sx + sa * sy
            ly = ca * sy - sa * sx
            msk = ((jnp.abs(sz) <= hz) & (jnp.abs(lx) <= hx)
                   & (jnp.abs(ly) <= hy))
            tot = plsc.all_reduce_population_count(msk)[0]
            off = jnp.minimum(cnt, CAP)
            plsc.store_compressed(compact_v.at[pl.ds(off, L)], iota + base,
                                  mask=msk)
            return cnt + tot

        cnt = lax.fori_loop(0, SCAN_STEPS, scan_body, jnp.int32(0),
                            unroll=4)

        denom = jnp.maximum(cnt, 1)
        ne_i = (cnt > 0).astype(jnp.int32)
        ne_f = (cnt > 0).astype(jnp.float32)
        boff = b * N

        def pick_body(g, _c):
            jv = iota + g * L
            sel = jv % denom
            lidx = plsc.load_gather(compact_v, [sel]) * ne_i
            plsc.store_scatter(pidx_v, [jv], lidx + boff, mask=mall)
            return _c

        lax.fori_loop(0, S // L, pick_body, jnp.int32(0))

        def fire_gather(h):
            s = h % 2
            return pltpu.async_copy(
                tablea_hbm.at[pidx_v.at[pl.ds(h * CH, CH)]],
                rows[s], gsem[s])

        def do_patch(hp):
            sp = hp % 2

            def patch_body(g, _c):
                jl = iota + g * L
                jv = jl + hp * CH
                sel = jv % denom
                lidx = plsc.load_gather(compact_v, [sel]) * ne_i
                px = plsc.load_gather(coords_v, [lidx])
                py = plsc.load_gather(coords_v, [lidx + N])
                pz = plsc.load_gather(coords_v, [lidx + 2 * N])
                rx = px - cx
                ry = py - cy
                ox = (rx * ca + ry * sa) * ne_f
                oy = (ry * ca - rx * sa) * ne_f
                oz = (pz - czb) * ne_f
                plsc.store_scatter(rows[sp], [jl, zi], ox, mask=mall)
                plsc.store_scatter(rows[sp], [jl, zi + 1], oy, mask=mall)
                plsc.store_scatter(rows[sp], [jl, zi + 2], oz, mask=mall)
                for c in range(3):
                    tv = plsc.load_gather(tailc_v, [lidx + c * N]) * ne_f
                    plsc.store_scatter(tail_v, [jl, zi + c], tv,
                                       mask=mall)
                return _c

            lax.fori_loop(0, CH // L, patch_body, jnp.int32(0))

            @pl.when(cnt == 0)
            def _zero_rows():
                zfull = jnp.zeros((L,), jnp.float32)

                def zr(i, _c):
                    for c8 in range(C // L):
                        rows[sp][i, pl.ds(c8 * L, L)] = zfull
                    return _c
                lax.fori_loop(0, CH, zr, jnp.int32(0))

            wm = pltpu.async_copy(
                rows[sp],
                out_hbm.at[m, pl.ds(hp * CH, CH), pl.ds(0, C)], wsem)
            wt = pltpu.async_copy(
                tail_v,
                out_hbm.at[m, pl.ds(hp * CH, CH), pl.ds(C, 3)], wsem)
            return [wm, wt]

        pending_w = {0: [], 1: []}
        g_prev = fire_gather(0)
        for h in range(1, NCH + 1):
            s = h % 2
            for cp in pending_w[s]:
                cp.wait()
            pending_w[s] = []
            g_new = fire_gather(h) if h < NCH else None
            g_prev.wait()
            pending_w[(h - 1) % 2] = do_patch(h - 1)
            g_prev = g_new
        for s in range(2):
            for cp in pending_w[s]:
                cp.wait()
        return _carry

    lax.fori_loop(0, BOXES_PER_W, box_body, jnp.int32(0))


@jax.jit
def _run_sc(coords_flat, tailc_flat, boxtab_flat, tablea):
    mesh = plsc.VectorSubcoreMesh(core_axis_name="c", subcore_axis_name="s")
    f = pl.kernel(
        _sc_body,
        out_type=jax.ShapeDtypeStruct((M_TOTAL, S, D), jnp.float32),
        mesh=mesh,
        scratch_types=[
            pltpu.VMEM((3 * N,), jnp.float32),
            pltpu.VMEM((3 * N,), jnp.float32),
            pltpu.VMEM((BOXES_PER_W * 16,), jnp.float32),
            pltpu.VMEM((CAP + L,), jnp.int32),
            pltpu.VMEM((S,), jnp.int32),
            pltpu.VMEM((CH, DP), jnp.float32),
            pltpu.VMEM((CH, DP), jnp.float32),
            pltpu.VMEM((CH, 3), jnp.float32),
            pltpu.SemaphoreType.DMA,
            pltpu.SemaphoreType.DMA,
            pltpu.SemaphoreType.DMA,
        ],
        compiler_params=pltpu.CompilerParams(needs_layout_passes=False),
    )
    return f(coords_flat, tailc_flat, boxtab_flat, tablea)


def kernel(feats, coordinate, batch_inds, rois):
    del batch_inds
    B = coordinate.shape[0]
    r = rois[:, 1:]
    cx, cy, cz = r[:, 0], r[:, 1], r[:, 2]
    dx, dy, dz = r[:, 3], r[:, 4], r[:, 5]
    rz = r[:, 6]
    zcol = jnp.zeros_like(cx)
    boxtab = jnp.stack(
        [cx, cy, cz, cz + dz / 2.0, dx / 2.0, dy / 2.0, dz / 2.0,
         jnp.cos(rz), jnp.sin(rz), zcol, zcol, zcol, zcol, zcol, zcol, zcol],
        axis=-1)
    coords_flat = coordinate.transpose(0, 2, 1).reshape(B * 3 * N)
    tailc_flat = feats[:, :, C - 3:].transpose(0, 2, 1).reshape(B * 3 * N)
    tablea = jnp.concatenate([coordinate, feats[:, :, :C - 3]],
                             axis=-1).reshape(B * N, DP)
    return _run_sc(coords_flat, tailc_flat, boxtab.reshape(-1), tablea)


# speedup vs baseline: 1.3168x; 1.0090x over previous
import jax
import jax.numpy as jnp
from jax import lax
from jax.experimental import pallas as pl
from jax.experimental.pallas import tpu as pltpu
from jax.experimental.pallas import tpu_sc as plsc

N = 16384
M_TOTAL = 256
S = 512
C = 128
D = C + 3
DP = 128
L = 16
NC = 2
NS = 16
NW = NC * NS
BOXES_PER_W = M_TOTAL // NW
SCAN_STEPS = N // L
CAP = S
CH = 64
NCH = S // CH


def _sc_body(coords_hbm, tailc_hbm, boxtab_hbm, tablea_hbm, out_hbm,
             coords_v, tailc_v, boxtab_v, compact_v, pidx_v,
             rows0_v, rows1_v, tail_v, gsem0, gsem1, wsem):
    wid = lax.axis_index("s") * NC + lax.axis_index("c")
    b = wid // NS
    pltpu.sync_copy(coords_hbm.at[pl.ds(b * 3 * N, 3 * N)], coords_v)
    pltpu.sync_copy(tailc_hbm.at[pl.ds(b * 3 * N, 3 * N)], tailc_v)
    pltpu.sync_copy(boxtab_hbm.at[pl.ds(wid * BOXES_PER_W * 16,
                                        BOXES_PER_W * 16)], boxtab_v)
    iota = lax.iota(jnp.int32, L)
    zi = jnp.zeros((L,), jnp.int32)
    mall = iota >= 0
    rows = (rows0_v, rows1_v)
    gsem = (gsem0, gsem1)

    def box_body(k, _carry):
        m = wid * BOXES_PER_W + k
        row_v = boxtab_v[pl.ds(k * 16, 16)]
        cx = row_v[0]
        cy = row_v[1]
        czb = row_v[2]
        czc = row_v[3]
        hx = row_v[4]
        hy = row_v[5]
        hz = row_v[6]
        ca = row_v[7]
        sa = row_v[8]

        compact_v[pl.ds(0, L)] = zi

        def scan_body(i, cnt):
            base = i * L
            xv = coords_v[pl.ds(base, L)]
            yv = coords_v[pl.ds(N + base, L)]
            zv = coords_v[pl.ds(2 * N + base, L)]
            sx = xv - cx
            sy = yv - cy
            sz = zv - czc
            lx = ca * sx + sa * sy
            ly = ca * sy - sa * sx
            msk = ((jnp.abs(sz) <= hz) & (jnp.abs(lx) <= hx)
                   & (jnp.abs(ly) <= hy))
            tot = plsc.all_reduce_population_count(msk)[0]
            off = jnp.minimum(cnt, CAP)
            plsc.store_compressed(compact_v.at[pl.ds(off, L)], iota + base,
                                  mask=msk)
            return cnt + tot

        cnt = lax.fori_loop(0, SCAN_STEPS, scan_body, jnp.int32(0),
                            unroll=8)

        denom = jnp.maximum(cnt, 1)
        ne_i = (cnt > 0).astype(jnp.int32)
        ne_f = (cnt > 0).astype(jnp.float32)
        boff = b * N

        def pick_body(g, _c):
            jv = iota + g * L
            sel = jv % denom
            lidx = plsc.load_gather(compact_v, [sel]) * ne_i
            plsc.store_scatter(pidx_v, [jv], lidx + boff, mask=mall)
            return _c

        lax.fori_loop(0, S // L, pick_body, jnp.int32(0))

        def fire_gather(h):
            s = h % 2
            return pltpu.async_copy(
                tablea_hbm.at[pidx_v.at[pl.ds(h * CH, CH)]],
                rows[s], gsem[s])

        def do_patch(hp):
            sp = hp % 2

            def patch_body(g, _c):
                jl = iota + g * L
                jv = jl + hp * CH
                sel = jv % denom
                lidx = plsc.load_gather(compact_v, [sel]) * ne_i
                px = plsc.load_gather(coords_v, [lidx])
                py = plsc.load_gather(coords_v, [lidx + N])
                pz = plsc.load_gather(coords_v, [lidx + 2 * N])
                rx = px - cx
                ry = py - cy
                ox = (rx * ca + ry * sa) * ne_f
                oy = (ry * ca - rx * sa) * ne_f
                oz = (pz - czb) * ne_f
                plsc.store_scatter(rows[sp], [jl, zi], ox, mask=mall)
                plsc.store_scatter(rows[sp], [jl, zi + 1], oy, mask=mall)
                plsc.store_scatter(rows[sp], [jl, zi + 2], oz, mask=mall)
                for c in range(3):
                    tv = plsc.load_gather(tailc_v, [lidx + c * N]) * ne_f
                    plsc.store_scatter(tail_v, [jl, zi + c], tv,
                                       mask=mall)
                return _c

            lax.fori_loop(0, CH // L, patch_body, jnp.int32(0))

            @pl.when(cnt == 0)
            def _zero_rows():
                zfull = jnp.zeros((L,), jnp.float32)

                def zr(i, _c):
                    for c8 in range(C // L):
                        rows[sp][i, pl.ds(c8 * L, L)] = zfull
                    return _c
                lax.fori_loop(0, CH, zr, jnp.int32(0))

            wm = pltpu.async_copy(
                rows[sp],
                out_hbm.at[m, pl.ds(hp * CH, CH), pl.ds(0, C)], wsem)
            wt = pltpu.async_copy(
                tail_v,
                out_hbm.at[m, pl.ds(hp * CH, CH), pl.ds(C, 3)], wsem)
            return [wm, wt]

        pending_w = {0: [], 1: []}
        g_prev = fire_gather(0)
        for h in range(1, NCH + 1):
            s = h % 2
            for cp in pending_w[s]:
                cp.wait()
            pending_w[s] = []
            g_new = fire_gather(h) if h < NCH else None
            g_prev.wait()
            pending_w[(h - 1) % 2] = do_patch(h - 1)
            g_prev = g_new
        for s in range(2):
            for cp in pending_w[s]:
                cp.wait()
        return _carry

    lax.fori_loop(0, BOXES_PER_W, box_body, jnp.int32(0))


@jax.jit
def _run_sc(coords_flat, tailc_flat, boxtab_flat, tablea):
    mesh = plsc.VectorSubcoreMesh(core_axis_name="c", subcore_axis_name="s")
    f = pl.kernel(
        _sc_body,
        out_type=jax.ShapeDtypeStruct((M_TOTAL, S, D), jnp.float32),
        mesh=mesh,
        scratch_types=[
            pltpu.VMEM((3 * N,), jnp.float32),
            pltpu.VMEM((3 * N,), jnp.float32),
            pltpu.VMEM((BOXES_PER_W * 16,), jnp.float32),
            pltpu.VMEM((CAP + L,), jnp.int32),
            pltpu.VMEM((S,), jnp.int32),
            pltpu.VMEM((CH, DP), jnp.float32),
            pltpu.VMEM((CH, DP), jnp.float32),
            pltpu.VMEM((CH, 3), jnp.float32),
            pltpu.SemaphoreType.DMA,
            pltpu.SemaphoreType.DMA,
            pltpu.SemaphoreType.DMA,
        ],
        compiler_params=pltpu.CompilerParams(needs_layout_passes=False),
    )
    return f(coords_flat, tailc_flat, boxtab_flat, tablea)


def kernel(feats, coordinate, batch_inds, rois):
    del batch_inds
    B = coordinate.shape[0]
    r = rois[:, 1:]
    cx, cy, cz = r[:, 0], r[:, 1], r[:, 2]
    dx, dy, dz = r[:, 3], r[:, 4], r[:, 5]
    rz = r[:, 6]
    zcol = jnp.zeros_like(cx)
    boxtab = jnp.stack(
        [cx, cy, cz, cz + dz / 2.0, dx / 2.0, dy / 2.0, dz / 2.0,
         jnp.cos(rz), jnp.sin(rz), zcol, zcol, zcol, zcol, zcol, zcol, zcol],
        axis=-1)
    coords_flat = coordinate.transpose(0, 2, 1).reshape(B * 3 * N)
    tailc_flat = feats[:, :, C - 3:].transpose(0, 2, 1).reshape(B * 3 * N)
    tablea = jnp.concatenate([coordinate, feats[:, :, :C - 3]],
                             axis=-1).reshape(B * N, DP)
    return _run_sc(coords_flat, tailc_flat, boxtab.reshape(-1), tablea)
